# Initial kernel scaffold; baseline (speedup 1.0000x reference)
import jax
import jax.numpy as jnp
from jax.experimental import pallas as pl


def kernel(x, edge_index, W, b):
    raise NotImplementedError("write your pallas kernel here")


# speedup vs baseline: 3.1617x; 3.1617x over previous
import functools

import jax
import jax.numpy as jnp
from jax import lax
from jax.experimental import pallas as pl
from jax.experimental.pallas import tpu as pltpu
from jax.experimental.pallas import tpu_sc as plsc

N_NODES = 10000
N_EDGES = 320000
D = 128

NC = 2
NS = 16
NW = NC * NS
K = 128

PW = ((N_EDGES // NW) + K - 1) // K * K
CH = PW // K
EPAD = PW * NW
NPAD = 10240
RPS = NPAD // NS

_mesh = plsc.VectorSubcoreMesh(core_axis_name="c", subcore_axis_name="s")


@functools.partial(
    pl.kernel,
    out_type=jax.ShapeDtypeStruct((NC, 2, NPAD), jnp.float32),
    mesh=_mesh,
    scratch_types=[
        pltpu.VMEM((K,), jnp.int32),
        pltpu.VMEM((K,), jnp.int32),
        pltpu.VMEM((K,), jnp.float32),
        pltpu.VMEM_SHARED((NPAD,), jnp.float32),
        pltpu.VMEM_SHARED((NPAD,), jnp.float32),
    ],
)
def _hist(src_hbm, dst_hbm, zrow_hbm, out_hbm, sidx, didx, ones_v, dego_sh, degi_sh):
    cid = lax.axis_index("c")
    sid = lax.axis_index("s")
    wid = sid * NC + cid
    pltpu.sync_copy(zrow_hbm.at[pl.ds(0, RPS)], dego_sh.at[pl.ds(sid * RPS, RPS)])
    pltpu.sync_copy(zrow_hbm.at[pl.ds(0, RPS)], degi_sh.at[pl.ds(sid * RPS, RPS)])
    for i in range(K // 16):
        ones_v[pl.ds(i * 16, 16)] = jnp.ones((16,), jnp.float32)
    plsc.subcore_barrier()

    def body(j, _):
        base = wid * PW + j * K
        pltpu.sync_copy(src_hbm.at[pl.ds(base, K)], sidx)
        pltpu.sync_copy(dst_hbm.at[pl.ds(base, K)], didx)
        pltpu.sync_copy(ones_v, dego_sh.at[sidx], add=True)
        pltpu.sync_copy(ones_v, degi_sh.at[didx], add=True)
        return 0

    lax.fori_loop(0, CH, body, 0)
    plsc.subcore_barrier()
    pltpu.sync_copy(dego_sh.at[pl.ds(sid * RPS, RPS)],
                    out_hbm.at[cid, 0, pl.ds(sid * RPS, RPS)])
    pltpu.sync_copy(degi_sh.at[pl.ds(sid * RPS, RPS)],
                    out_hbm.at[cid, 1, pl.ds(sid * RPS, RPS)])


@functools.partial(
    pl.kernel,
    out_type=jax.ShapeDtypeStruct((NC, NPAD, D), jnp.float32),
    mesh=_mesh,
    scratch_types=[
        pltpu.VMEM((K,), jnp.int32),
        pltpu.VMEM((K,), jnp.int32),
        pltpu.VMEM((K, D), jnp.float32),
        pltpu.VMEM_SHARED((NPAD, D), jnp.float32),
        pltpu.SemaphoreType.DMA,
    ],
)
def _agg(h_hbm, src_hbm, dst_hbm, zrows_hbm, out_hbm, sidx, didx, rows_v, acc_sh, sem):
    cid = lax.axis_index("c")
    sid = lax.axis_index("s")
    wid = sid * NC + cid
    pltpu.sync_copy(zrows_hbm, acc_sh.at[pl.ds(sid * RPS, RPS)])
    plsc.subcore_barrier()

    def body(j, _):
        base = wid * PW + j * K
        pltpu.sync_copy(src_hbm.at[pl.ds(base, K)], sidx)
        pltpu.sync_copy(dst_hbm.at[pl.ds(base, K)], didx)
        pltpu.async_copy(h_hbm.at[sidx], rows_v, sem).wait()
        pltpu.sync_copy(rows_v, acc_sh.at[didx], add=True)
        return 0

    lax.fori_loop(0, CH, body, 0)
    plsc.subcore_barrier()
    pltpu.sync_copy(acc_sh.at[pl.ds(sid * RPS, RPS)],
                    out_hbm.at[cid, pl.ds(sid * RPS, RPS)])


def _h_body(x_ref, degp_ref, h_ref):
    deg = degp_ref[:, 0] + degp_ref[:, 1]
    norm = lax.rsqrt(jnp.maximum(deg, 1.0))
    h_ref[...] = x_ref[...] * norm[:, None]


def _final_body(x_ref, aggp_ref, degp_ref, w_ref, b_ref, o_ref):
    agg = aggp_ref[0] + aggp_ref[1]
    deg = degp_ref[:, 0] + degp_ref[:, 1]
    norm = lax.rsqrt(jnp.maximum(deg, 1.0))
    rst = jnp.dot(agg * norm[:, None], w_ref[...],
                  preferred_element_type=jnp.float32)
    o_ref[...] = x_ref[...] + rst + b_ref[...]


def kernel(x, edge_index, W, b):
    N, d = x.shape
    E = edge_index.shape[1]
    pad_val = jnp.int32(N)
    src_p = jnp.concatenate(
        [edge_index[0].astype(jnp.int32), jnp.full((EPAD - E,), pad_val)])
    dst_p = jnp.concatenate(
        [edge_index[1].astype(jnp.int32), jnp.full((EPAD - E,), pad_val)])
    zrow = jnp.zeros((RPS,), jnp.float32)
    zrows = jnp.zeros((RPS, D), jnp.float32)

    degs = _hist(src_p, dst_p, zrow)
    dego_p = degs[:, 0, :].T
    degi_p = degs[:, 1, :].T

    x_pad = jnp.pad(x, ((0, NPAD - N), (0, 0)))
    HB = 512
    h_pad = pl.pallas_call(
        _h_body,
        grid=(NPAD // HB,),
        in_specs=[
            pl.BlockSpec((HB, D), lambda i: (i, 0)),
            pl.BlockSpec((HB, NC), lambda i: (i, 0)),
        ],
        out_specs=pl.BlockSpec((HB, D), lambda i: (i, 0)),
        out_shape=jax.ShapeDtypeStruct((NPAD, D), jnp.float32),
    )(x_pad, dego_p)

    aggp = _agg(h_pad, src_p, dst_p, zrows)

    FB = 400
    out = pl.pallas_call(
        _final_body,
        grid=(N // FB,),
        in_specs=[
            pl.BlockSpec((FB, D), lambda i: (i, 0)),
            pl.BlockSpec((NC, FB, D), lambda i: (0, i, 0)),
            pl.BlockSpec((FB, NC), lambda i: (i, 0)),
            pl.BlockSpec((D, D), lambda i: (0, 0)),
            pl.BlockSpec((1, D), lambda i: (0, 0)),
        ],
        out_specs=pl.BlockSpec((FB, D), lambda i: (i, 0)),
        out_shape=jax.ShapeDtypeStruct((N, D), jnp.float32),
    )(x, aggp, degi_p, W, b.reshape(1, D))
    return out
